# baseline (speedup 1.0000x reference)
import functools

import jax
import jax.numpy as jnp
from jax import lax
from jax.experimental import pallas as pl
from jax.experimental.pallas import tpu as pltpu
from jax.experimental.pallas import tpu_sc as plsc

VOCAB = 100000
N_DIM = 768
BATCH = 4
SEQ = 8192

NUM_CORES = 2
NUM_SUBCORES = 16
NUM_WORKERS = NUM_CORES * NUM_SUBCORES

LANES = 16
N_ROWS = BATCH * SEQ
ROWS_PER_WORKER = N_ROWS // NUM_WORKERS
CHUNK = 64
NUM_CHUNKS = ROWS_PER_WORKER // CHUNK
VECS_PER_ROW = N_DIM // LANES


def _make_sc_kernel():
  mesh = plsc.VectorSubcoreMesh(
      core_axis_name="c", subcore_axis_name="s", num_cores=NUM_CORES
  )

  @functools.partial(
      pl.kernel,
      out_type=jax.ShapeDtypeStruct((N_ROWS, N_DIM), jnp.float32),
      mesh=mesh,
      scratch_types=[
          pltpu.VMEM((CHUNK,), jnp.int32),
          pltpu.VMEM((CHUNK, N_DIM), jnp.float32),
          pltpu.VMEM((CHUNK, N_DIM), jnp.float32),
          pltpu.SemaphoreType.DMA,
      ],
  )
  def sc_kernel(table_hbm, tokens_hbm, pos_hbm, out_hbm, idx_v, buf_v, pos_v,
                sem):
    wid = lax.axis_index("s") * NUM_CORES + lax.axis_index("c")
    base = wid * ROWS_PER_WORKER
    pos_base = base % SEQ

    def do_chunk(g, _):
      row0 = base + g * CHUNK
      p0 = pos_base + g * CHUNK
      pltpu.sync_copy(tokens_hbm.at[pl.ds(row0, CHUNK)], idx_v)
      gather = pltpu.async_copy(table_hbm.at[idx_v], buf_v, sem)
      pltpu.sync_copy(pos_hbm.at[pl.ds(p0, CHUNK)], pos_v)
      gather.wait()

      def add_row(i, _):
        for j in range(VECS_PER_ROW):
          sl = pl.ds(j * LANES, LANES)
          plsc.addupdate(buf_v.at[i, sl], pos_v[i, sl])
        return 0

      lax.fori_loop(0, CHUNK, add_row, 0)
      pltpu.sync_copy(buf_v, out_hbm.at[pl.ds(row0, CHUNK)])
      return 0

    lax.fori_loop(0, NUM_CHUNKS, do_chunk, 0)

  return sc_kernel


_sc_kernel = _make_sc_kernel()


@jax.jit
def kernel(tokens, token_table, pos_table):
  tokens_flat = tokens.reshape(-1).astype(jnp.int32)
  out = _sc_kernel(token_table, tokens_flat, pos_table)
  return out.reshape(BATCH, SEQ, N_DIM)


# speedup vs baseline: 1.3767x; 1.3767x over previous
import functools

import jax
import jax.numpy as jnp
from jax import lax
from jax.experimental import pallas as pl
from jax.experimental.pallas import tpu as pltpu
from jax.experimental.pallas import tpu_sc as plsc

VOCAB = 100000
N_DIM = 768
BATCH = 4
SEQ = 8192

NUM_CORES = 2
NUM_SUBCORES = 16
NUM_WORKERS = NUM_CORES * NUM_SUBCORES

LANES = 16
VECS_PER_ROW = N_DIM // LANES
S_PER_WORKER = SEQ // NUM_WORKERS
CHUNK = 32
POS_CHUNKS = S_PER_WORKER // CHUNK
NWORK = 3
SLOT = [0, 1, 2, 0]


def _make_sc_kernel():
  mesh = plsc.VectorSubcoreMesh(
      core_axis_name="c", subcore_axis_name="s", num_cores=NUM_CORES
  )

  @functools.partial(
      pl.kernel,
      out_type=jax.ShapeDtypeStruct((BATCH * SEQ, N_DIM), jnp.float32),
      mesh=mesh,
      scratch_types=[
          pltpu.VMEM((BATCH, S_PER_WORKER), jnp.int32),
          [pltpu.VMEM((CHUNK, N_DIM), jnp.float32)] * NWORK,
          pltpu.VMEM((CHUNK, N_DIM), jnp.float32),
          [pltpu.SemaphoreType.DMA] * NWORK,
          [pltpu.SemaphoreType.DMA] * NWORK,
          pltpu.SemaphoreType.DMA,
      ],
  )
  def sc_kernel(table_hbm, tokens_hbm, pos_hbm, out_hbm,
                idx_v, work, pos_v, gsem, wsem, psem):
    wid = lax.axis_index("s") * NUM_CORES + lax.axis_index("c")
    s0 = wid * S_PER_WORKER

    for b in range(BATCH):
      pltpu.sync_copy(
          tokens_hbm.at[pl.ds(b * SEQ + s0, S_PER_WORKER)], idx_v.at[b]
      )

    def add_pos(buf):
      def add_row(i, _):
        for j in range(VECS_PER_ROW):
          sl = pl.ds(j * LANES, LANES)
          plsc.addupdate(buf.at[i, sl], pos_v[i, sl])
        return 0
      lax.fori_loop(0, CHUNK, add_row, 0)

    def do_chunk(pc, _):
      def gather(b):
        w = SLOT[b]
        return pltpu.async_copy(
            table_hbm.at[idx_v.at[b, pl.ds(pc * CHUNK, CHUNK)]], work[w],
            gsem[w],
        )

      def writeback(b):
        w = SLOT[b]
        row0 = b * SEQ + s0 + pc * CHUNK
        return pltpu.async_copy(
            work[w], out_hbm.at[pl.ds(row0, CHUNK)], wsem[w]
        )

      posd = pltpu.async_copy(
          pos_hbm.at[pl.ds(s0 + pc * CHUNK, CHUNK)], pos_v, psem
      )
      g0 = gather(0)
      g1 = gather(1)
      g2 = gather(2)
      posd.wait()
      g0.wait(); add_pos(work[0]); w0 = writeback(0)
      g1.wait(); add_pos(work[1]); w1 = writeback(1)
      w0.wait()
      g3 = gather(3)
      g2.wait(); add_pos(work[2]); w2 = writeback(2)
      g3.wait(); add_pos(work[0]); w3 = writeback(3)
      w1.wait(); w2.wait(); w3.wait()
      return 0

    lax.fori_loop(0, POS_CHUNKS, do_chunk, 0)

  return sc_kernel


_sc_kernel = _make_sc_kernel()


@jax.jit
def kernel(tokens, token_table, pos_table):
  tokens_flat = tokens.reshape(-1).astype(jnp.int32)
  out = _sc_kernel(token_table, tokens_flat, pos_table)
  return out.reshape(BATCH, SEQ, N_DIM)


# speedup vs baseline: 1.5149x; 1.1004x over previous
import functools

import jax
import jax.numpy as jnp
from jax import lax
from jax.experimental import pallas as pl
from jax.experimental.pallas import tpu as pltpu
from jax.experimental.pallas import tpu_sc as plsc

VOCAB = 100000
N_DIM = 768
BATCH = 4
SEQ = 8192

NUM_CORES = 2
NUM_SUBCORES = 16
NUM_WORKERS = NUM_CORES * NUM_SUBCORES

LANES = 16
VECS_PER_ROW = N_DIM // LANES
S_PER_WORKER = SEQ // NUM_WORKERS
CHUNK = 16
POS_CHUNKS = S_PER_WORKER // CHUNK
LAST = POS_CHUNKS - 1
NWORK = 4
NPOS = 2


def _make_sc_kernel():
  mesh = plsc.VectorSubcoreMesh(
      core_axis_name="c", subcore_axis_name="s", num_cores=NUM_CORES
  )

  @functools.partial(
      pl.kernel,
      out_type=jax.ShapeDtypeStruct((BATCH * SEQ, N_DIM), jnp.float32),
      mesh=mesh,
      scratch_types=[
          pltpu.VMEM((BATCH, S_PER_WORKER), jnp.int32),
          [pltpu.VMEM((CHUNK, N_DIM), jnp.float32)] * NWORK,
          [pltpu.VMEM((CHUNK, N_DIM), jnp.float32)] * NPOS,
          [pltpu.SemaphoreType.DMA] * NWORK,
          [pltpu.SemaphoreType.DMA] * NWORK,
          [pltpu.SemaphoreType.DMA] * NPOS,
      ],
  )
  def sc_kernel(table_hbm, tokens_hbm, pos_hbm, out_hbm,
                idx_v, work, posb, gsem, wsem, psem):
    wid = lax.axis_index("s") * NUM_CORES + lax.axis_index("c")
    s0 = wid * S_PER_WORKER

    for b in range(BATCH):
      pltpu.sync_copy(
          tokens_hbm.at[pl.ds(b * SEQ + s0, S_PER_WORKER)], idx_v.at[b]
      )

    def add_pos(buf, pv):
      def add_row(i, _):
        for j in range(VECS_PER_ROW):
          sl = pl.ds(j * LANES, LANES)
          plsc.addupdate(buf.at[i, sl], pv[i, sl])
        return 0
      lax.fori_loop(0, CHUNK, add_row, 0)

    def issue_pos(pc, ps):
      return pltpu.async_copy(
          pos_hbm.at[pl.ds(s0 + pc * CHUNK, CHUNK)], posb[ps], psem[ps]
      )

    def issue_gather(pc, b):
      return pltpu.async_copy(
          table_hbm.at[idx_v.at[b, pl.ds(pc * CHUNK, CHUNK)]], work[b],
          gsem[b],
      )

    def issue_write(pc, b):
      row0 = b * SEQ + s0 + pc * CHUNK
      return pltpu.async_copy(work[b], out_hbm.at[pl.ds(row0, CHUNK)], wsem[b])

    def wait_gather(b):
      pltpu.make_async_copy(
          table_hbm.at[pl.ds(0, CHUNK)], work[b], gsem[b]
      ).wait()

    def wait_write(b):
      pltpu.make_async_copy(
          work[b], out_hbm.at[pl.ds(0, CHUNK)], wsem[b]
      ).wait()

    def chunk_step(pc, ps):
      @pl.when(pc > 0)
      def _():
        wait_write(3)
      issue_gather(pc, 3)

      wait_gather(0)
      pltpu.make_async_copy(
          pos_hbm.at[pl.ds(0, CHUNK)], posb[ps], psem[ps]
      ).wait()

      @pl.when(pc < LAST)
      def _():
        issue_pos(pc + 1, 1 - ps)
      add_pos(work[0], posb[ps])
      issue_write(pc, 0)

      wait_gather(1)
      add_pos(work[1], posb[ps])
      issue_write(pc, 1)

      wait_write(0)

      @pl.when(pc < LAST)
      def _():
        issue_gather(pc + 1, 0)

      wait_gather(2)
      add_pos(work[2], posb[ps])
      issue_write(pc, 2)

      wait_write(1)

      @pl.when(pc < LAST)
      def _():
        issue_gather(pc + 1, 1)

      wait_gather(3)
      add_pos(work[3], posb[ps])
      issue_write(pc, 3)

      wait_write(2)

      @pl.when(pc < LAST)
      def _():
        issue_gather(pc + 1, 2)

    issue_pos(0, 0)
    for b in range(3):
      issue_gather(0, b)

    def body(k, _):
      chunk_step(2 * k, 0)
      chunk_step(2 * k + 1, 1)
      return 0

    lax.fori_loop(0, POS_CHUNKS // 2, body, 0)
    wait_write(3)

  return sc_kernel


_sc_kernel = _make_sc_kernel()


@jax.jit
def kernel(tokens, token_table, pos_table):
  tokens_flat = tokens.reshape(-1).astype(jnp.int32)
  out = _sc_kernel(token_table, tokens_flat, pos_table)
  return out.reshape(BATCH, SEQ, N_DIM)


# speedup vs baseline: 1.7208x; 1.1359x over previous
import functools

import jax
import jax.numpy as jnp
from jax import lax
from jax.experimental import pallas as pl
from jax.experimental.pallas import tpu as pltpu
from jax.experimental.pallas import tpu_sc as plsc

VOCAB = 100000
N_DIM = 768
BATCH = 4
SEQ = 8192

NUM_CORES = 2
NUM_SUBCORES = 16
NUM_WORKERS = NUM_CORES * NUM_SUBCORES

LANES = 16
VECS_PER_ROW = N_DIM // LANES
S_PER_WORKER = SEQ // NUM_WORKERS
CHUNK = 16
POS_CHUNKS = S_PER_WORKER // CHUNK
LAST = POS_CHUNKS - 1
NWORK = 4
NPOS = 2


def _make_sc_kernel():
  mesh = plsc.VectorSubcoreMesh(
      core_axis_name="c", subcore_axis_name="s", num_cores=NUM_CORES
  )

  @functools.partial(
      pl.kernel,
      out_type=jax.ShapeDtypeStruct((BATCH * SEQ, N_DIM), jnp.float32),
      mesh=mesh,
      scratch_types=[
          pltpu.VMEM((BATCH, S_PER_WORKER), jnp.int32),
          [pltpu.VMEM((CHUNK, N_DIM), jnp.float32)] * NWORK,
          [pltpu.VMEM((CHUNK, N_DIM), jnp.float32)] * NPOS,
          [pltpu.SemaphoreType.DMA] * NWORK,
          [pltpu.SemaphoreType.DMA] * NWORK,
          [pltpu.SemaphoreType.DMA] * NPOS,
      ],
  )
  def sc_kernel(table_hbm, tokens_hbm, pos_hbm, out_hbm,
                idx_v, work, posb, gsem, wsem, psem):
    wid = lax.axis_index("s") * NUM_CORES + lax.axis_index("c")
    s0 = wid * S_PER_WORKER

    for b in range(BATCH):
      pltpu.sync_copy(
          tokens_hbm.at[pl.ds(b * SEQ + s0, S_PER_WORKER)], idx_v.at[b]
      )

    def add_pos(buf, pv):
      @plsc.parallel_loop(0, CHUNK, 1, unroll=2)
      def _(i):
        for j in range(VECS_PER_ROW):
          sl = pl.ds(j * LANES, LANES)
          plsc.addupdate(buf.at[i, sl], pv[i, sl])

    def issue_pos(pc, ps):
      return pltpu.async_copy(
          pos_hbm.at[pl.ds(s0 + pc * CHUNK, CHUNK)], posb[ps], psem[ps]
      )

    def issue_gather(pc, b):
      return pltpu.async_copy(
          table_hbm.at[idx_v.at[b, pl.ds(pc * CHUNK, CHUNK)]], work[b],
          gsem[b],
      )

    def issue_write(pc, b):
      row0 = b * SEQ + s0 + pc * CHUNK
      return pltpu.async_copy(work[b], out_hbm.at[pl.ds(row0, CHUNK)], wsem[b])

    def wait_gather(b):
      pltpu.make_async_copy(
          table_hbm.at[pl.ds(0, CHUNK)], work[b], gsem[b]
      ).wait()

    def wait_write(b):
      pltpu.make_async_copy(
          work[b], out_hbm.at[pl.ds(0, CHUNK)], wsem[b]
      ).wait()

    def chunk_step(pc, ps):
      @pl.when(pc > 0)
      def _():
        wait_write(3)
      issue_gather(pc, 3)

      wait_gather(0)
      pltpu.make_async_copy(
          pos_hbm.at[pl.ds(0, CHUNK)], posb[ps], psem[ps]
      ).wait()

      @pl.when(pc < LAST)
      def _():
        issue_pos(pc + 1, 1 - ps)
      add_pos(work[0], posb[ps])
      issue_write(pc, 0)

      wait_gather(1)
      add_pos(work[1], posb[ps])
      issue_write(pc, 1)

      wait_write(0)

      @pl.when(pc < LAST)
      def _():
        issue_gather(pc + 1, 0)

      wait_gather(2)
      add_pos(work[2], posb[ps])
      issue_write(pc, 2)

      wait_write(1)

      @pl.when(pc < LAST)
      def _():
        issue_gather(pc + 1, 1)

      wait_gather(3)
      add_pos(work[3], posb[ps])
      issue_write(pc, 3)

      wait_write(2)

      @pl.when(pc < LAST)
      def _():
        issue_gather(pc + 1, 2)

    issue_pos(0, 0)
    for b in range(3):
      issue_gather(0, b)

    def body(k, _):
      chunk_step(2 * k, 0)
      chunk_step(2 * k + 1, 1)
      return 0

    lax.fori_loop(0, POS_CHUNKS // 2, body, 0)
    wait_write(3)

  return sc_kernel


_sc_kernel = _make_sc_kernel()


@jax.jit
def kernel(tokens, token_table, pos_table):
  tokens_flat = tokens.reshape(-1).astype(jnp.int32)
  out = _sc_kernel(token_table, tokens_flat, pos_table)
  return out.reshape(BATCH, SEQ, N_DIM)
